# baseline (speedup 1.0000x reference)
import dataclasses
import functools

import jax
import jax.numpy as jnp
from jax import lax
from jax.experimental import pallas as pl
from jax.experimental.pallas import tpu as pltpu
from jax.experimental.pallas import tpu_sc as plsc

_TOKENS = 4 * 8192
_CH = 1024

_SC_TOKENS = 16384
_TC_TOKENS = _TOKENS - _SC_TOKENS

_TC_BLK = 1024
_TC_DATA_STEPS = _TC_TOKENS // _TC_BLK
_TC_STEPS = _TOKENS // _TC_BLK

_NC, _NS, _L = 2, 16, 16
_NW = _NC * _NS
_W = _SC_TOKENS // _NW
_G = 16


def _tc_kernel(p_ref, t_ref, m_ref, sq_ref, cnt_ref):
    i = pl.program_id(0)

    @pl.when(i == 0)
    def _init():
        sq_ref[0, 0] = 0.0
        cnt_ref[0, 0] = 0.0

    m = m_ref[0, 0]
    cnt_ref[0, 0] += jnp.sum(m)

    @pl.when(i < _TC_DATA_STEPS)
    def _data():
        d = p_ref[...] - t_ref[...]
        row_sq = jnp.sum(d * d, axis=1)
        sq_ref[0, 0] += jnp.sum(row_sq * m)


def _tc_call(pred, tgt, m_full):
    return pl.pallas_call(
        _tc_kernel,
        grid=(_TC_STEPS,),
        in_specs=[
            pl.BlockSpec((_TC_BLK, _CH),
                         lambda i: (jnp.minimum(i, _TC_DATA_STEPS - 1), 0)),
            pl.BlockSpec((_TC_BLK, _CH),
                         lambda i: (jnp.minimum(i, _TC_DATA_STEPS - 1), 0)),
            pl.BlockSpec((1, 1, _TC_BLK), lambda i: (i, 0, 0)),
        ],
        out_specs=[
            pl.BlockSpec(memory_space=pltpu.SMEM),
            pl.BlockSpec(memory_space=pltpu.SMEM),
        ],
        out_shape=[
            jax.ShapeDtypeStruct((1, 1), jnp.float32),
            jax.ShapeDtypeStruct((1, 1), jnp.float32),
        ],
    )(pred, tgt, m_full)


def _sc_call(pred, tgt, m_i32):
    mesh = plsc.VectorSubcoreMesh(core_axis_name="c", subcore_axis_name="s")
    cp = pltpu.CompilerParams()
    if "needs_layout_passes" in pltpu.CompilerParams.__dataclass_fields__:
        cp = dataclasses.replace(cp, needs_layout_passes=False)

    @functools.partial(
        pl.kernel,
        mesh=mesh,
        compiler_params=cp,
        out_type=jax.ShapeDtypeStruct((_NC, _NS, _L), jnp.float32),
        scratch_types=[
            pltpu.VMEM((_L,), jnp.float32),
            pltpu.VMEM((_W,), jnp.int32),
            pltpu.VMEM((_W + _G,), jnp.int32),
            pltpu.VMEM((_G, _CH), jnp.float32),
            pltpu.VMEM((_G, _CH), jnp.float32),
            pltpu.VMEM((_G, _CH), jnp.float32),
            pltpu.VMEM((_G, _CH), jnp.float32),
            pltpu.SemaphoreType.DMA,
            pltpu.SemaphoreType.DMA,
            pltpu.SemaphoreType.DMA,
            pltpu.SemaphoreType.DMA,
            pltpu.SemaphoreType.DMA,
        ],
    )
    def sc_kernel(p_hbm, t_hbm, m_hbm, o_hbm, acc_ref, m_v, idx_v,
                  pb0, pb1, tb0, tb1, sem_m, sp0, sp1, st0, st1):
        c = lax.axis_index("c")
        s = lax.axis_index("s")
        wid = s * _NC + c
        tok0 = _TC_TOKENS + wid * _W

        acc_ref[...] = jnp.zeros((_L,), jnp.float32)
        pltpu.make_async_copy(m_hbm.at[pl.ds(tok0, _W)], m_v, sem_m).start()
        pltpu.make_async_copy(m_hbm.at[pl.ds(tok0, _W)], m_v, sem_m).wait()

        def _compact(g, n):
            mvec = m_v[pl.ds(g * _L, _L)]
            msk = mvec != 0
            x = lax.iota(jnp.int32, _L) + (tok0 + g * _L)
            plsc.store_compressed(idx_v.at[pl.ds(n, _L)], x, mask=msk)
            return n + plsc.all_reduce_population_count(msk)[0]

        n = lax.fori_loop(0, _W // _L, _compact, jnp.int32(0))
        idx_v[pl.ds(n, _G)] = jnp.full((_G,), tok0, jnp.int32)

        nb = (n + _G - 1) // _G

        def _start(b, pb, tb, sp, st):
            sl = idx_v.at[pl.ds(b * _G, _G)]
            pltpu.make_async_copy(p_hbm.at[sl], pb, sp).start()
            pltpu.make_async_copy(t_hbm.at[sl], tb, st).start()

        def _compute(b, pb, tb):
            @pl.loop(0, _G)
            def _row(r):
                zero = jnp.zeros((_L,), jnp.float32)

                def _chunk4(c4, carry):
                    a0, a1, a2, a3 = carry
                    base = c4 * (4 * _L)
                    d0 = pb[r, pl.ds(base, _L)] - tb[r, pl.ds(base, _L)]
                    d1 = (pb[r, pl.ds(base + _L, _L)]
                          - tb[r, pl.ds(base + _L, _L)])
                    d2 = (pb[r, pl.ds(base + 2 * _L, _L)]
                          - tb[r, pl.ds(base + 2 * _L, _L)])
                    d3 = (pb[r, pl.ds(base + 3 * _L, _L)]
                          - tb[r, pl.ds(base + 3 * _L, _L)])
                    return (a0 + d0 * d0, a1 + d1 * d1,
                            a2 + d2 * d2, a3 + d3 * d3)

                a0, a1, a2, a3 = lax.fori_loop(
                    0, _CH // (4 * _L), _chunk4, (zero, zero, zero, zero))
                tmp = (a0 + a1) + (a2 + a3)
                w = jnp.where(b * _G + r < n, 1.0, 0.0).astype(jnp.float32)
                acc_ref[...] += tmp * w

        @pl.when(nb > 0)
        def _go():
            _start(0, pb0, tb0, sp0, st0)

            def _pair(i2, carry):
                b = i2 * 2

                @pl.when(b + 1 < nb)
                def _s1():
                    _start(b + 1, pb1, tb1, sp1, st1)

                pltpu.make_async_copy(p_hbm.at[idx_v.at[pl.ds(0, _G)]],
                                      pb0, sp0).wait()
                pltpu.make_async_copy(t_hbm.at[idx_v.at[pl.ds(0, _G)]],
                                      tb0, st0).wait()
                _compute(b, pb0, tb0)

                @pl.when(b + 2 < nb)
                def _s2():
                    _start(b + 2, pb0, tb0, sp0, st0)

                @pl.when(b + 1 < nb)
                def _c1():
                    pltpu.make_async_copy(p_hbm.at[idx_v.at[pl.ds(0, _G)]],
                                          pb1, sp1).wait()
                    pltpu.make_async_copy(t_hbm.at[idx_v.at[pl.ds(0, _G)]],
                                          tb1, st1).wait()
                    _compute(b + 1, pb1, tb1)

                return carry

            lax.fori_loop(0, (nb + 1) // 2, _pair, jnp.int32(0))

        pltpu.sync_copy(acc_ref, o_hbm.at[c, s])

    return sc_kernel(pred, tgt, m_i32)


def kernel(predicted, target, mask):
    tgt_dim = target.shape[-1]
    pred = predicted[..., :tgt_dim].reshape(_TOKENS, _CH)
    tgt = target.reshape(_TOKENS, _CH)
    m_flat = mask.reshape(_TOKENS)
    m_i32 = m_flat.astype(jnp.int32)
    m_full = m_flat.astype(jnp.float32).reshape(_TC_STEPS, 1, _TC_BLK)

    sc_part = _sc_call(pred, tgt, m_i32)
    sq_tc, cnt = _tc_call(pred, tgt, m_full)

    total_sq = sq_tc[0, 0] + jnp.sum(sc_part)
    n = cnt[0, 0] * _CH
    return total_sq / jnp.maximum(n, 1.0)


# speedup vs baseline: 1.1624x; 1.1624x over previous
import jax
import jax.numpy as jnp
from jax.experimental import pallas as pl
from jax.experimental.pallas import tpu as pltpu

_TOKENS = 4 * 8192
_CH = 1024
_BLK_T = 1024
_GRID = _TOKENS // _BLK_T


def _masked_mse_kernel(p_ref, t_ref, m_ref, loss_ref, acc_ref, cnt_ref):
    i = pl.program_id(0)

    @pl.when(i == 0)
    def _init():
        acc_ref[0] = 0.0
        cnt_ref[0] = 0.0

    d = p_ref[...] - t_ref[...]
    m = m_ref[0, 0]
    row_sq = jnp.sum(d * d, axis=1)
    acc_ref[0] += jnp.sum(row_sq * m)
    cnt_ref[0] += jnp.sum(m)

    @pl.when(i == _GRID - 1)
    def _fin():
        n = cnt_ref[0] * _CH
        loss_ref[0, 0] = acc_ref[0] / jnp.maximum(n, 1.0)


def kernel(predicted, target, mask):
    tgt_dim = target.shape[-1]
    pred = predicted[..., :tgt_dim].reshape(_TOKENS, _CH)
    tgt = target.reshape(_TOKENS, _CH)
    m = mask.reshape(_GRID, 1, _BLK_T).astype(jnp.float32)

    loss = pl.pallas_call(
        _masked_mse_kernel,
        grid=(_GRID,),
        in_specs=[
            pl.BlockSpec((_BLK_T, _CH), lambda i: (i, 0)),
            pl.BlockSpec((_BLK_T, _CH), lambda i: (i, 0)),
            pl.BlockSpec((1, 1, _BLK_T), lambda i: (i, 0, 0)),
        ],
        out_specs=pl.BlockSpec(memory_space=pltpu.SMEM),
        out_shape=jax.ShapeDtypeStruct((1, 1), jnp.float32),
        scratch_shapes=[
            pltpu.SMEM((1,), jnp.float32),
            pltpu.SMEM((1,), jnp.float32),
        ],
    )(pred, tgt, m)
    return loss[0, 0]
